# Initial kernel scaffold; baseline (speedup 1.0000x reference)
import jax
import jax.numpy as jnp
from jax.experimental import pallas as pl


def kernel(sentences_seq, position_to_entity1_batch, position_to_entity2_batch, word_embedding, position_embedding):
    raise NotImplementedError("write your pallas kernel here")


# speedup vs baseline: 4.5447x; 4.5447x over previous
import functools

import jax
import jax.numpy as jnp
from jax import lax
from jax.experimental import pallas as pl
from jax.experimental.pallas import tpu as pltpu
from jax.experimental.pallas import tpu_sc as plsc

D_MODEL = 64
SEQ = 200
BATCH = 4096
ROWS = SEQ * BATCH
NC = 2
NS = 16
NW = NC * NS
PER_W = ROWS // NW
K = 128
NCHUNK = PER_W // K
LANES = 16


def _emb_body(seq_hbm, p1_hbm, p2_hbm, wtab_hbm, ptab_hbm, out_hbm,
              idx_w, idx_p1, idx_p2, buf_w, buf_p1, buf_p2, sem):
    wid = lax.axis_index("s") * NC + lax.axis_index("c")
    base = wid * PER_W

    def chunk(i, _):
        off = base + i * K
        pltpu.sync_copy(seq_hbm.at[pl.ds(off, K)], idx_w)
        pltpu.sync_copy(p1_hbm.at[pl.ds(off, K)], idx_p1)
        pltpu.sync_copy(p2_hbm.at[pl.ds(off, K)], idx_p2)
        cw = pltpu.async_copy(wtab_hbm.at[idx_w], buf_w, sem)
        c1 = pltpu.async_copy(ptab_hbm.at[idx_p1], buf_p1, sem)
        c2 = pltpu.async_copy(ptab_hbm.at[idx_p2], buf_p2, sem)
        cw.wait()
        c1.wait()
        c2.wait()

        def row(j, _):
            for k in range(D_MODEL // LANES):
                sl = pl.ds(k * LANES, LANES)
                buf_w[j, sl] = buf_w[j, sl] + buf_p1[j, sl] + buf_p2[j, sl]
            return 0

        lax.fori_loop(0, K, row, 0, unroll=2)
        pltpu.sync_copy(buf_w, out_hbm.at[pl.ds(off, K)])
        return 0

    lax.fori_loop(0, NCHUNK, chunk, 0)


@jax.jit
def _run(seq_flat, p1_flat, p2_flat, wtab, ptab):
    mesh = plsc.VectorSubcoreMesh(
        core_axis_name="c", subcore_axis_name="s",
        num_cores=NC, num_subcores=NS)
    return pl.kernel(
        _emb_body,
        out_type=jax.ShapeDtypeStruct((ROWS, D_MODEL), jnp.float32),
        mesh=mesh,
        scratch_types=[
            pltpu.VMEM((K,), jnp.int32),
            pltpu.VMEM((K,), jnp.int32),
            pltpu.VMEM((K,), jnp.int32),
            pltpu.VMEM((K, D_MODEL), jnp.float32),
            pltpu.VMEM((K, D_MODEL), jnp.float32),
            pltpu.VMEM((K, D_MODEL), jnp.float32),
            pltpu.SemaphoreType.DMA,
        ],
        compiler_params=pltpu.CompilerParams(use_tc_tiling_on_sc=False),
    )(seq_flat, p1_flat, p2_flat, wtab, ptab)


def kernel(sentences_seq, position_to_entity1_batch, position_to_entity2_batch,
           word_embedding, position_embedding):
    seq_flat = sentences_seq.T.reshape(-1).astype(jnp.int32)
    p1_flat = position_to_entity1_batch.T.reshape(-1).astype(jnp.int32)
    p2_flat = position_to_entity2_batch.T.reshape(-1).astype(jnp.int32)
    out = _run(seq_flat, p1_flat, p2_flat,
               word_embedding.astype(jnp.float32),
               position_embedding.astype(jnp.float32))
    return out.reshape(BATCH, SEQ, D_MODEL)


# speedup vs baseline: 5.3381x; 1.1746x over previous
import functools

import jax
import jax.numpy as jnp
from jax import lax
from jax.experimental import pallas as pl
from jax.experimental.pallas import tpu as pltpu
from jax.experimental.pallas import tpu_sc as plsc

D_MODEL = 64
SEQ = 200
BATCH = 4096
ROWS = SEQ * BATCH
NC = 2
NS = 16
NW = NC * NS
PER_W = ROWS // NW
K = 128
NCHUNK = PER_W // K
LANES = 16


def _emb_body(seq_hbm, p1_hbm, p2_hbm, wtab_hbm, ptab_hbm, out_hbm,
              idx_w, idx_p1, idx_p2, buf_w, buf_p1, buf_p2, sem):
    wid = lax.axis_index("s") * NC + lax.axis_index("c")
    base = wid * PER_W

    def chunk(i, _):
        off = base + i * K
        pltpu.sync_copy(seq_hbm.at[pl.ds(off, K)], idx_w)
        pltpu.sync_copy(p1_hbm.at[pl.ds(off, K)], idx_p1)
        pltpu.sync_copy(p2_hbm.at[pl.ds(off, K)], idx_p2)
        cw = pltpu.async_copy(wtab_hbm.at[idx_w], buf_w, sem)
        cw.wait()
        c1 = pltpu.async_copy(ptab_hbm.at[idx_p1], buf_w, sem, add=True)
        c1.wait()
        c2 = pltpu.async_copy(ptab_hbm.at[idx_p2], buf_w, sem, add=True)
        c2.wait()
        pltpu.sync_copy(buf_w, out_hbm.at[pl.ds(off, K)])
        return 0

    lax.fori_loop(0, NCHUNK, chunk, 0)


@jax.jit
def _run(seq_flat, p1_flat, p2_flat, wtab, ptab):
    mesh = plsc.VectorSubcoreMesh(
        core_axis_name="c", subcore_axis_name="s",
        num_cores=NC, num_subcores=NS)
    return pl.kernel(
        _emb_body,
        out_type=jax.ShapeDtypeStruct((ROWS, D_MODEL), jnp.float32),
        mesh=mesh,
        scratch_types=[
            pltpu.VMEM((K,), jnp.int32),
            pltpu.VMEM((K,), jnp.int32),
            pltpu.VMEM((K,), jnp.int32),
            pltpu.VMEM((K, D_MODEL), jnp.float32),
            pltpu.VMEM((K, D_MODEL), jnp.float32),
            pltpu.VMEM((K, D_MODEL), jnp.float32),
            pltpu.SemaphoreType.DMA,
        ],
        compiler_params=pltpu.CompilerParams(use_tc_tiling_on_sc=False),
    )(seq_flat, p1_flat, p2_flat, wtab, ptab)


def kernel(sentences_seq, position_to_entity1_batch, position_to_entity2_batch,
           word_embedding, position_embedding):
    seq_flat = sentences_seq.T.reshape(-1).astype(jnp.int32)
    p1_flat = position_to_entity1_batch.T.reshape(-1).astype(jnp.int32)
    p2_flat = position_to_entity2_batch.T.reshape(-1).astype(jnp.int32)
    out = _run(seq_flat, p1_flat, p2_flat,
               word_embedding.astype(jnp.float32),
               position_embedding.astype(jnp.float32))
    return out.reshape(BATCH, SEQ, D_MODEL)


# speedup vs baseline: 6.5952x; 1.2355x over previous
import functools

import jax
import jax.numpy as jnp
from jax import lax
from jax.experimental import pallas as pl
from jax.experimental.pallas import tpu as pltpu
from jax.experimental.pallas import tpu_sc as plsc

D_MODEL = 64
SEQ = 200
BATCH = 4096
ROWS = SEQ * BATCH
NC = 2
NS = 16
NW = NC * NS
PER_W = ROWS // NW
KC = 128
NCH = PER_W // KC
NB = 4


def _emb_body(seq_hbm, p1_hbm, p2_hbm, wtab_hbm, ptab_hbm, out_hbm,
              idx_w, idx_p1, idx_p2, b0, b1, b2, b3, sems):
    wid = lax.axis_index("s") * NC + lax.axis_index("c")
    rowbase = wid * NCH
    base = wid * PER_W

    pltpu.sync_copy(seq_hbm.at[pl.ds(rowbase, NCH)], idx_w)
    pltpu.sync_copy(p1_hbm.at[pl.ds(rowbase, NCH)], idx_p1)
    pltpu.sync_copy(p2_hbm.at[pl.ds(rowbase, NCH)], idx_p2)

    bufs = [b0, b1, b2, b3]

    def drain(b):
        pltpu.make_async_copy(out_hbm.at[pl.ds(0, KC)], bufs[b],
                              sems.at[b]).wait()

    def step(t, b):
        @pl.when((t >= 3) & (t < NCH + 3))
        def _():
            b3 = (b - 3) % NB
            drain(b3)
            pltpu.async_copy(bufs[b3],
                             out_hbm.at[pl.ds(base + (t - 3) * KC, KC)],
                             sems.at[b3])

        @pl.when((t >= 2) & (t < NCH + 2))
        def _():
            b2_ = (b - 2) % NB
            drain(b2_)
            pltpu.async_copy(ptab_hbm.at[idx_p2.at[t - 2]], bufs[b2_],
                             sems.at[b2_], add=True)

        @pl.when((t >= 1) & (t < NCH + 1))
        def _():
            b1_ = (b - 1) % NB
            drain(b1_)
            pltpu.async_copy(ptab_hbm.at[idx_p1.at[t - 1]], bufs[b1_],
                             sems.at[b1_], add=True)

        @pl.when(t < NCH)
        def _():
            @pl.when(t >= NB)
            def _():
                drain(b)

            pltpu.async_copy(wtab_hbm.at[idx_w.at[t]], bufs[b], sems.at[b])

    def outer(g, _):
        for b in range(NB):
            step(g * NB + b, b)
        return 0

    lax.fori_loop(0, (NCH + 3 + NB - 1) // NB, outer, 0)

    for b in range(NB):
        drain(b)


@jax.jit
def _run(seq2d, p1_2d, p2_2d, wtab, ptab):
    mesh = plsc.VectorSubcoreMesh(
        core_axis_name="c", subcore_axis_name="s",
        num_cores=NC, num_subcores=NS)
    return pl.kernel(
        _emb_body,
        out_type=jax.ShapeDtypeStruct((ROWS, D_MODEL), jnp.float32),
        mesh=mesh,
        scratch_types=[
            pltpu.VMEM((NCH, KC), jnp.int32),
            pltpu.VMEM((NCH, KC), jnp.int32),
            pltpu.VMEM((NCH, KC), jnp.int32),
            pltpu.VMEM((KC, D_MODEL), jnp.float32),
            pltpu.VMEM((KC, D_MODEL), jnp.float32),
            pltpu.VMEM((KC, D_MODEL), jnp.float32),
            pltpu.VMEM((KC, D_MODEL), jnp.float32),
            pltpu.SemaphoreType.DMA((NB,)),
        ],
        compiler_params=pltpu.CompilerParams(use_tc_tiling_on_sc=False),
    )(seq2d, p1_2d, p2_2d, wtab, ptab)


def kernel(sentences_seq, position_to_entity1_batch, position_to_entity2_batch,
           word_embedding, position_embedding):
    seq2d = sentences_seq.T.reshape(ROWS // KC, KC).astype(jnp.int32)
    p1_2d = position_to_entity1_batch.T.reshape(ROWS // KC, KC).astype(jnp.int32)
    p2_2d = position_to_entity2_batch.T.reshape(ROWS // KC, KC).astype(jnp.int32)
    out = _run(seq2d, p1_2d, p2_2d,
               word_embedding.astype(jnp.float32),
               position_embedding.astype(jnp.float32))
    return out.reshape(BATCH, SEQ, D_MODEL)


# speedup vs baseline: 6.6314x; 1.0055x over previous
import functools

import jax
import jax.numpy as jnp
from jax import lax
from jax.experimental import pallas as pl
from jax.experimental.pallas import tpu as pltpu
from jax.experimental.pallas import tpu_sc as plsc

D_MODEL = 64
SEQ = 200
BATCH = 4096
ROWS = SEQ * BATCH
NC = 2
NS = 16
NW = NC * NS
PER_W = ROWS // NW
KC = 256
SPLIT = 2
IDXW = KC // SPLIT
NB = 4
NHALF = 2
NCH_H = PER_W // KC // NHALF


def _emb_body(seq_hbm, p1_hbm, p2_hbm, wtab_hbm, ptab_hbm, out_hbm,
              idx_w, idx_p1, idx_p2, b0, b1, b2, b3, sems):
    wid = lax.axis_index("s") * NC + lax.axis_index("c")
    bufs = [b0, b1, b2, b3]

    def drain(b):
        pltpu.make_async_copy(out_hbm.at[pl.ds(0, KC)], bufs[b],
                              sems.at[b]).wait()

    for h in range(NHALF):
        idxrow = (wid * NHALF + h) * NCH_H * SPLIT
        base = (wid * NHALF + h) * NCH_H * KC
        pltpu.sync_copy(seq_hbm.at[pl.ds(idxrow, NCH_H * SPLIT)], idx_w)
        pltpu.sync_copy(p1_hbm.at[pl.ds(idxrow, NCH_H * SPLIT)], idx_p1)
        pltpu.sync_copy(p2_hbm.at[pl.ds(idxrow, NCH_H * SPLIT)], idx_p2)

        def step(t, b):
            @pl.when((t >= 3) & (t < NCH_H + 3))
            def _():
                b3 = (b - 3) % NB
                drain(b3)
                pltpu.async_copy(bufs[b3],
                                 out_hbm.at[pl.ds(base + (t - 3) * KC, KC)],
                                 sems.at[b3])

            @pl.when((t >= 2) & (t < NCH_H + 2))
            def _():
                b2_ = (b - 2) % NB
                drain(b2_)
                for q in range(SPLIT):
                    pltpu.async_copy(
                        ptab_hbm.at[idx_p2.at[(t - 2) * SPLIT + q]],
                        bufs[b2_].at[pl.ds(q * IDXW, IDXW)],
                        sems.at[b2_], add=True)

            @pl.when((t >= 1) & (t < NCH_H + 1))
            def _():
                b1_ = (b - 1) % NB
                drain(b1_)
                for q in range(SPLIT):
                    pltpu.async_copy(
                        ptab_hbm.at[idx_p1.at[(t - 1) * SPLIT + q]],
                        bufs[b1_].at[pl.ds(q * IDXW, IDXW)],
                        sems.at[b1_], add=True)

            @pl.when(t < NCH_H)
            def _():
                @pl.when(t >= NB)
                def _():
                    drain(b)

                for q in range(SPLIT):
                    pltpu.async_copy(
                        wtab_hbm.at[idx_w.at[t * SPLIT + q]],
                        bufs[b].at[pl.ds(q * IDXW, IDXW)],
                        sems.at[b])

        def outer(g, _):
            for b in range(NB):
                step(g * NB + b, b)
            return 0

        lax.fori_loop(0, (NCH_H + 3 + NB - 1) // NB, outer, 0)

        for b in range(NB):
            drain(b)


@jax.jit
def _run(seq2d, p1_2d, p2_2d, wtab, ptab):
    mesh = plsc.VectorSubcoreMesh(
        core_axis_name="c", subcore_axis_name="s",
        num_cores=NC, num_subcores=NS)
    return pl.kernel(
        _emb_body,
        out_type=jax.ShapeDtypeStruct((ROWS, D_MODEL), jnp.float32),
        mesh=mesh,
        scratch_types=[
            pltpu.VMEM((NCH_H * SPLIT, IDXW), jnp.int32),
            pltpu.VMEM((NCH_H * SPLIT, IDXW), jnp.int32),
            pltpu.VMEM((NCH_H * SPLIT, IDXW), jnp.int32),
            pltpu.VMEM((KC, D_MODEL), jnp.float32),
            pltpu.VMEM((KC, D_MODEL), jnp.float32),
            pltpu.VMEM((KC, D_MODEL), jnp.float32),
            pltpu.VMEM((KC, D_MODEL), jnp.float32),
            pltpu.SemaphoreType.DMA((NB,)),
        ],
        compiler_params=pltpu.CompilerParams(use_tc_tiling_on_sc=False),
    )(seq2d, p1_2d, p2_2d, wtab, ptab)


def kernel(sentences_seq, position_to_entity1_batch, position_to_entity2_batch,
           word_embedding, position_embedding):
    seq2d = sentences_seq.T.reshape(ROWS // IDXW, IDXW).astype(jnp.int32)
    p1_2d = position_to_entity1_batch.T.reshape(ROWS // IDXW, IDXW).astype(jnp.int32)
    p2_2d = position_to_entity2_batch.T.reshape(ROWS // IDXW, IDXW).astype(jnp.int32)
    out = _run(seq2d, p1_2d, p2_2d,
               word_embedding.astype(jnp.float32),
               position_embedding.astype(jnp.float32))
    return out.reshape(BATCH, SEQ, D_MODEL)


# speedup vs baseline: 6.9801x; 1.0526x over previous
import functools

import jax
import jax.numpy as jnp
from jax import lax
from jax.experimental import pallas as pl
from jax.experimental.pallas import tpu as pltpu
from jax.experimental.pallas import tpu_sc as plsc

D_MODEL = 64
SEQ = 200
BATCH = 4096
ROWS = SEQ * BATCH
NC = 2
NS = 16
NW = NC * NS
PER_W = ROWS // NW
KC = 256
SPLIT = 2
IDXW = KC // SPLIT
NB = 4
NHALF = 2
NCH_H = PER_W // KC // NHALF
LANES = 16


def _emb_body(seq_hbm, p1_hbm, p2_hbm, wtab_hbm, ptab_hbm, out_hbm,
              idx_w, idx_p1, idx_p2, idx_out, b0, b1, b2, b3, sems):
    wid = lax.axis_index("s") * NC + lax.axis_index("c")
    bufs = [b0, b1, b2, b3]
    iota = lax.iota(jnp.int32, LANES)

    def drain(b):
        pltpu.make_async_copy(out_hbm.at[pl.ds(0, KC)], bufs[b],
                              sems.at[b]).wait()

    for h in range(NHALF):
        idxrow = (wid * NHALF + h) * NCH_H * SPLIT
        base = (wid * NHALF + h) * NCH_H * KC
        pltpu.sync_copy(seq_hbm.at[pl.ds(idxrow, NCH_H * SPLIT)], idx_w)
        pltpu.sync_copy(p1_hbm.at[pl.ds(idxrow, NCH_H * SPLIT)], idx_p1)
        pltpu.sync_copy(p2_hbm.at[pl.ds(idxrow, NCH_H * SPLIT)], idx_p2)

        def step(t, b):
            @pl.when((t >= 3) & (t < NCH_H + 3))
            def _():
                b3 = (b - 3) % NB
                drain(b3)
                tok0 = base + (t - 3) * KC
                for q in range(SPLIT):
                    for v in range(IDXW // LANES):
                        tok = tok0 + q * IDXW + v * LANES + iota
                        orow = ((tok & (BATCH - 1)) * SEQ) + (tok >> 12)
                        idx_out[b3 * SPLIT + q, pl.ds(v * LANES, LANES)] = orow
                for q in range(SPLIT):
                    pltpu.async_copy(
                        bufs[b3].at[pl.ds(q * IDXW, IDXW)],
                        out_hbm.at[idx_out.at[b3 * SPLIT + q]],
                        sems.at[b3])

            @pl.when((t >= 2) & (t < NCH_H + 2))
            def _():
                b2_ = (b - 2) % NB
                drain(b2_)
                for q in range(SPLIT):
                    pltpu.async_copy(
                        ptab_hbm.at[idx_p2.at[(t - 2) * SPLIT + q]],
                        bufs[b2_].at[pl.ds(q * IDXW, IDXW)],
                        sems.at[b2_], add=True)

            @pl.when((t >= 1) & (t < NCH_H + 1))
            def _():
                b1_ = (b - 1) % NB
                drain(b1_)
                for q in range(SPLIT):
                    pltpu.async_copy(
                        ptab_hbm.at[idx_p1.at[(t - 1) * SPLIT + q]],
                        bufs[b1_].at[pl.ds(q * IDXW, IDXW)],
                        sems.at[b1_], add=True)

            @pl.when(t < NCH_H)
            def _():
                @pl.when(t >= NB)
                def _():
                    drain(b)

                for q in range(SPLIT):
                    pltpu.async_copy(
                        wtab_hbm.at[idx_w.at[t * SPLIT + q]],
                        bufs[b].at[pl.ds(q * IDXW, IDXW)],
                        sems.at[b])

        def outer(g, _):
            for b in range(NB):
                step(g * NB + b, b)
            return 0

        lax.fori_loop(0, (NCH_H + 3 + NB - 1) // NB, outer, 0)

        for b in range(NB):
            drain(b)


@jax.jit
def _run(seq2d, p1_2d, p2_2d, wtab, ptab):
    mesh = plsc.VectorSubcoreMesh(
        core_axis_name="c", subcore_axis_name="s",
        num_cores=NC, num_subcores=NS)
    return pl.kernel(
        _emb_body,
        out_type=jax.ShapeDtypeStruct((ROWS, D_MODEL), jnp.float32),
        mesh=mesh,
        scratch_types=[
            pltpu.VMEM((NCH_H * SPLIT, IDXW), jnp.int32),
            pltpu.VMEM((NCH_H * SPLIT, IDXW), jnp.int32),
            pltpu.VMEM((NCH_H * SPLIT, IDXW), jnp.int32),
            pltpu.VMEM((NB * SPLIT, IDXW), jnp.int32),
            pltpu.VMEM((KC, D_MODEL), jnp.float32),
            pltpu.VMEM((KC, D_MODEL), jnp.float32),
            pltpu.VMEM((KC, D_MODEL), jnp.float32),
            pltpu.VMEM((KC, D_MODEL), jnp.float32),
            pltpu.SemaphoreType.DMA((NB,)),
        ],
        compiler_params=pltpu.CompilerParams(use_tc_tiling_on_sc=False),
    )(seq2d, p1_2d, p2_2d, wtab, ptab)


def kernel(sentences_seq, position_to_entity1_batch, position_to_entity2_batch,
           word_embedding, position_embedding):
    seq2d = sentences_seq.reshape(ROWS // IDXW, IDXW).astype(jnp.int32)
    p1_2d = position_to_entity1_batch.reshape(ROWS // IDXW, IDXW).astype(jnp.int32)
    p2_2d = position_to_entity2_batch.reshape(ROWS // IDXW, IDXW).astype(jnp.int32)
    out = _run(seq2d, p1_2d, p2_2d,
               word_embedding.astype(jnp.float32),
               position_embedding.astype(jnp.float32))
    return out.reshape(BATCH, SEQ, D_MODEL)
